# baseline (speedup 1.0000x reference)
import functools

import jax
import jax.numpy as jnp
from jax import lax
from jax.experimental import pallas as pl
from jax.experimental.pallas import tpu as pltpu
from jax.experimental.pallas import tpu_sc as plsc

BATCH = 16384
EMBED = 32
NC = 2
NS = 16
NW = NC * NS
BPW = BATCH // NW
NCHUNK = 4
CHUNK = BPW // NCHUNK
GROUPS = BPW // 16


def _body(uids_hbm, mids_hbm, uembT_hbm, membT_hbm, ubiasT_hbm, mbiasT_hbm,
          out_hbm, uids_v, mids_v, ucols_v, mcols_v, ub_v, mb_v, out_v, sem):
    wid = lax.axis_index("s") * NC + lax.axis_index("c")
    base = wid * BPW

    for j in range(NCHUNK):
        pltpu.sync_copy(uids_hbm.at[pl.ds(base + j * CHUNK, CHUNK)],
                        uids_v.at[j])
        pltpu.sync_copy(mids_hbm.at[pl.ds(base + j * CHUNK, CHUNK)],
                        mids_v.at[j])

    def chunk_body(j, carry):
        lo = j * CHUNK
        copies = []
        for c in range(EMBED):
            copies.append(pltpu.async_copy(
                uembT_hbm.at[c].at[uids_v.at[j]],
                ucols_v.at[c, pl.ds(lo, CHUNK)], sem))
            copies.append(pltpu.async_copy(
                membT_hbm.at[c].at[mids_v.at[j]],
                mcols_v.at[c, pl.ds(lo, CHUNK)], sem))
        copies.append(pltpu.async_copy(
            ubiasT_hbm.at[0].at[uids_v.at[j]], ub_v.at[pl.ds(lo, CHUNK)], sem))
        copies.append(pltpu.async_copy(
            mbiasT_hbm.at[0].at[mids_v.at[j]], mb_v.at[pl.ds(lo, CHUNK)], sem))
        for cp in copies:
            cp.wait()
        return carry

    lax.fori_loop(0, NCHUNK, chunk_body, 0)

    def group(t, carry):
        b16 = t * 16
        acc = ub_v[pl.ds(b16, 16)] + mb_v[pl.ds(b16, 16)]
        for c in range(EMBED):
            acc = acc + ucols_v[c, pl.ds(b16, 16)] * mcols_v[c, pl.ds(b16, 16)]
        out_v[pl.ds(b16, 16)] = acc
        return carry

    lax.fori_loop(0, GROUPS, group, 0)

    pltpu.sync_copy(out_v, out_hbm.at[pl.ds(base, BPW)])


@jax.jit
def _run(uids, mids, uembT, membT, ubiasT, mbiasT):
    mesh = plsc.VectorSubcoreMesh(core_axis_name="c", subcore_axis_name="s")
    f = functools.partial(
        pl.kernel,
        mesh=mesh,
        compiler_params=pltpu.CompilerParams(
            needs_layout_passes=False, use_tc_tiling_on_sc=False),
        out_type=jax.ShapeDtypeStruct((BATCH,), jnp.float32),
        scratch_types=[
            pltpu.VMEM((NCHUNK, CHUNK), jnp.int32),
            pltpu.VMEM((NCHUNK, CHUNK), jnp.int32),
            pltpu.VMEM((EMBED, BPW), jnp.float32),
            pltpu.VMEM((EMBED, BPW), jnp.float32),
            pltpu.VMEM((BPW,), jnp.float32),
            pltpu.VMEM((BPW,), jnp.float32),
            pltpu.VMEM((BPW,), jnp.float32),
            pltpu.SemaphoreType.DMA,
        ],
    )(_body)
    return f(uids, mids, uembT, membT, ubiasT, mbiasT)


def kernel(user_ids, movie_ids, user_embedding, movie_embedding,
           user_bias, movie_bias):
    uids = user_ids.astype(jnp.int32)
    mids = movie_ids.astype(jnp.int32)
    uembT = jnp.transpose(user_embedding)
    membT = jnp.transpose(movie_embedding)
    ubiasT = jnp.transpose(user_bias)
    mbiasT = jnp.transpose(movie_bias)
    return _run(uids, mids, uembT, membT, ubiasT, mbiasT)
